# Initial kernel scaffold; baseline (speedup 1.0000x reference)
import jax
import jax.numpy as jnp
from jax.experimental import pallas as pl


def kernel(node_features, edge_features, from_idx, to_idx, enc_node_W, enc_node_b, enc_edge_W, enc_edge_b, msg_W1, msg_b1, msg_W2, msg_b2, rmsg_W1, rmsg_b1, rmsg_W2, rmsg_b2, node_W, node_b, fc1_W, fc1_b, fc2_W, fc2_b):
    raise NotImplementedError("write your pallas kernel here")


# speedup vs baseline: 2.6063x; 2.6063x over previous
import functools

import jax
import jax.numpy as jnp
from jax import lax
from jax.experimental import pallas as pl
from jax.experimental.pallas import tpu as pltpu
from jax.experimental.pallas import tpu_sc as plsc

N = 7680
E = 30720
D = 128
DE = 16
H = 128
TD = 64
MS = 15
NP = 256
NLAYERS = 3
TEMP = 0.1
SINK_ITERS = 20

E2 = 2 * E
NODE_BLK = 512
EDGE_BLK = 1024
GW = 128
CW = 128
PB = 16

_SC_CORES = 2
_SC_SUBCORES = 16
_SC_WORKERS = _SC_CORES * _SC_SUBCORES


def _enc_proj_body(nf, wenc, benc, wf, wt, h_out, p_out):
    h = jnp.dot(nf[...], wenc[...], preferred_element_type=jnp.float32) + benc[...]
    h_out[...] = h
    p_out[0] = jnp.dot(h, wf[...], preferred_element_type=jnp.float32)
    p_out[1] = jnp.dot(h, wt[...], preferred_element_type=jnp.float32)


def _encode_and_project(nf, wenc, benc, wf, wt):
    grid = (N // NODE_BLK,)
    return pl.pallas_call(
        _enc_proj_body,
        grid=grid,
        in_specs=[
            pl.BlockSpec((NODE_BLK, D), lambda i: (i, 0)),
            pl.BlockSpec((D, D), lambda i: (0, 0)),
            pl.BlockSpec((1, D), lambda i: (0, 0)),
            pl.BlockSpec((D, 2 * H), lambda i: (0, 0)),
            pl.BlockSpec((D, 2 * H), lambda i: (0, 0)),
        ],
        out_specs=[
            pl.BlockSpec((NODE_BLK, D), lambda i: (i, 0)),
            pl.BlockSpec((2, NODE_BLK, 2 * H), lambda i: (0, i, 0)),
        ],
        out_shape=[
            jax.ShapeDtypeStruct((N, D), jnp.float32),
            jax.ShapeDtypeStruct((2, N, 2 * H), jnp.float32),
        ],
    )(nf, wenc, benc, wf, wt)


def _edge_term_body(ef, wee, bee, wcc, bcc, ec_out):
    t = jnp.dot(ef[...], wee[...], preferred_element_type=jnp.float32) + bee[...]
    ec_out[...] = jnp.dot(t, wcc[...], preferred_element_type=jnp.float32) + bcc[...]


def _edge_term(ef, wee, bee, wcc, bcc):
    grid = (E // EDGE_BLK,)
    return pl.pallas_call(
        _edge_term_body,
        grid=grid,
        in_specs=[
            pl.BlockSpec((EDGE_BLK, DE), lambda i: (i, 0)),
            pl.BlockSpec((DE, DE), lambda i: (0, 0)),
            pl.BlockSpec((1, DE), lambda i: (0, 0)),
            pl.BlockSpec((DE, 2 * H), lambda i: (0, 0)),
            pl.BlockSpec((1, 2 * H), lambda i: (0, 0)),
        ],
        out_specs=pl.BlockSpec((EDGE_BLK, 2 * H), lambda i: (i, 0)),
        out_shape=jax.ShapeDtypeStruct((E, 2 * H), jnp.float32),
    )(ef, wee, bee, wcc, bcc)


def _msg_body(g, ec, w2, b2, rw2, rb2, m_out):
    x = jnp.maximum(g[0] + g[1] + ec[...], 0.0)
    m_out[0] = jnp.dot(x[:, :H], w2[...], preferred_element_type=jnp.float32) + b2[...]
    m_out[1] = jnp.dot(x[:, H:], rw2[...], preferred_element_type=jnp.float32) + rb2[...]


def _messages(g3, ec, w2, b2, rw2, rb2):
    grid = (E // EDGE_BLK,)
    return pl.pallas_call(
        _msg_body,
        grid=grid,
        in_specs=[
            pl.BlockSpec((2, EDGE_BLK, 2 * H), lambda i: (0, i, 0)),
            pl.BlockSpec((EDGE_BLK, 2 * H), lambda i: (i, 0)),
            pl.BlockSpec((H, H), lambda i: (0, 0)),
            pl.BlockSpec((1, H), lambda i: (0, 0)),
            pl.BlockSpec((H, H), lambda i: (0, 0)),
            pl.BlockSpec((1, H), lambda i: (0, 0)),
        ],
        out_specs=pl.BlockSpec((2, EDGE_BLK, H), lambda i: (0, i, 0)),
        out_shape=jax.ShapeDtypeStruct((2, E, H), jnp.float32),
    )(g3, ec, w2, b2, rw2, rb2)


def _update_proj_body(h, p, nwa, nwb, nb, wf, wt, h_out, p_out):
    agg = p[0] + p[1]
    hn = (jnp.dot(h[...], nwa[...], preferred_element_type=jnp.float32)
          + jnp.dot(agg, nwb[...], preferred_element_type=jnp.float32)
          + nb[...])
    h_out[...] = hn
    p_out[0] = jnp.dot(hn, wf[...], preferred_element_type=jnp.float32)
    p_out[1] = jnp.dot(hn, wt[...], preferred_element_type=jnp.float32)


def _update_and_project(h, parts, nwa, nwb, nb, wf, wt):
    grid = (N // NODE_BLK,)
    return pl.pallas_call(
        _update_proj_body,
        grid=grid,
        in_specs=[
            pl.BlockSpec((NODE_BLK, D), lambda i: (i, 0)),
            pl.BlockSpec((2, NODE_BLK, H), lambda i: (0, i, 0)),
            pl.BlockSpec((D, D), lambda i: (0, 0)),
            pl.BlockSpec((H, D), lambda i: (0, 0)),
            pl.BlockSpec((1, D), lambda i: (0, 0)),
            pl.BlockSpec((D, 2 * H), lambda i: (0, 0)),
            pl.BlockSpec((D, 2 * H), lambda i: (0, 0)),
        ],
        out_specs=[
            pl.BlockSpec((NODE_BLK, D), lambda i: (i, 0)),
            pl.BlockSpec((2, NODE_BLK, 2 * H), lambda i: (0, i, 0)),
        ],
        out_shape=[
            jax.ShapeDtypeStruct((N, D), jnp.float32),
            jax.ShapeDtypeStruct((2, N, 2 * H), jnp.float32),
        ],
    )(h, parts, nwa, nwb, nb, wf, wt)


def _update_final_body(h, p, nwa, nwb, nb, h_out):
    agg = p[0] + p[1]
    h_out[...] = (jnp.dot(h[...], nwa[...], preferred_element_type=jnp.float32)
                  + jnp.dot(agg, nwb[...], preferred_element_type=jnp.float32)
                  + nb[...])


def _update_final(h, parts, nwa, nwb, nb):
    grid = (N // NODE_BLK,)
    return pl.pallas_call(
        _update_final_body,
        grid=grid,
        in_specs=[
            pl.BlockSpec((NODE_BLK, D), lambda i: (i, 0)),
            pl.BlockSpec((2, NODE_BLK, H), lambda i: (0, i, 0)),
            pl.BlockSpec((D, D), lambda i: (0, 0)),
            pl.BlockSpec((H, D), lambda i: (0, 0)),
            pl.BlockSpec((1, D), lambda i: (0, 0)),
        ],
        out_specs=pl.BlockSpec((NODE_BLK, D), lambda i: (i, 0)),
        out_shape=jax.ShapeDtypeStruct((N, D), jnp.float32),
    )(h, parts, nwa, nwb, nb)


def _transform_body(h, w1, b1, w2, b2, t_out):
    t = jnp.maximum(jnp.dot(h[...], w1[...], preferred_element_type=jnp.float32) + b1[...], 0.0)
    t_out[...] = jnp.dot(t, w2[...], preferred_element_type=jnp.float32) + b2[...]


def _transform(h, w1, b1, w2, b2):
    grid = (N // NODE_BLK,)
    return pl.pallas_call(
        _transform_body,
        grid=grid,
        in_specs=[
            pl.BlockSpec((NODE_BLK, D), lambda i: (i, 0)),
            pl.BlockSpec((D, TD), lambda i: (0, 0)),
            pl.BlockSpec((1, TD), lambda i: (0, 0)),
            pl.BlockSpec((TD, TD), lambda i: (0, 0)),
            pl.BlockSpec((1, TD), lambda i: (0, 0)),
        ],
        out_specs=pl.BlockSpec((NODE_BLK, TD), lambda i: (i, 0)),
        out_shape=jax.ShapeDtypeStruct((N, TD), jnp.float32),
    )(h, w1, b1, w2, b2)


def _sinkhorn_body(x, t, out, s_ref, pc_ref):
    xb = x[...]
    tb = t[...]
    hq = xb[:, :MS, :]
    hc = xb[:, MS:, :]
    tq = tb[:, :MS, :]
    tct = jnp.swapaxes(tb[:, MS:, :], 1, 2)
    for b in range(PB):
        s_ref[b] = jnp.dot(tq[b], tct[b], preferred_element_type=jnp.float32)
    la = s_ref[...] * (1.0 / TEMP)

    def _iter(_, la):
        m2 = jnp.max(la, axis=2, keepdims=True)
        la = la - (m2 + jnp.log(jnp.sum(jnp.exp(la - m2), axis=2, keepdims=True)))
        m1 = jnp.max(la, axis=1, keepdims=True)
        la = la - (m1 + jnp.log(jnp.sum(jnp.exp(la - m1), axis=1, keepdims=True)))
        return la

    la = lax.fori_loop(0, SINK_ITERS, _iter, la)
    plan = jnp.exp(la)
    for b in range(PB):
        pc_ref[b] = jnp.dot(plan[b], hc[b], preferred_element_type=jnp.float32)
    diff = jnp.maximum(hq - pc_ref[...], 0.0)
    r = jnp.sum(jnp.sum(diff, axis=2), axis=1)
    out[...] = (-r).reshape(1, 1, PB)


def _sinkhorn_scores(x3, t3):
    grid = (NP // PB,)
    return pl.pallas_call(
        _sinkhorn_body,
        grid=grid,
        in_specs=[
            pl.BlockSpec((PB, 2 * MS, D), lambda i: (i, 0, 0)),
            pl.BlockSpec((PB, 2 * MS, TD), lambda i: (i, 0, 0)),
        ],
        out_specs=pl.BlockSpec((1, 1, PB), lambda i: (i, 0, 0)),
        out_shape=jax.ShapeDtypeStruct((NP // PB, 1, PB), jnp.float32),
        scratch_shapes=[
            pltpu.VMEM((PB, MS, MS), jnp.float32),
            pltpu.VMEM((PB, MS, D), jnp.float32),
        ],
    )(x3, t3)


def _sc_gather(table, idx2d):
    ni = idx2d.shape[1]
    cols = table.shape[1]
    mesh = plsc.VectorSubcoreMesh(core_axis_name="c", subcore_axis_name="s")

    @functools.partial(
        pl.kernel,
        out_type=jax.ShapeDtypeStruct((ni, cols), jnp.float32),
        mesh=mesh,
    )
    def k(tab_hbm, i_hbm, o_hbm):
        def body(i_vmem, o_vmem):
            pltpu.sync_copy(tab_hbm.at[i_vmem.at[0]], o_vmem)

        pltpu.emit_pipeline(
            body,
            grid=(ni // GW,),
            in_specs=[pl.BlockSpec((1, GW), lambda i: (0, i))],
            out_specs=[pl.BlockSpec((GW, cols), lambda i: (i, 0))],
            core_axis_name=("c", "s"),
            dimension_semantics=(pltpu.PARALLEL,),
        )(i_hbm, o_hbm)

    return k(table, idx2d)


def _sc_scatter_add(m2, idx2d, zeros_nd):
    nch = idx2d.shape[0]
    ch_per_core = nch // _SC_CORES
    ch_per_worker = nch // _SC_WORKERS
    rows_per_sub = N // _SC_SUBCORES
    mesh = plsc.VectorSubcoreMesh(core_axis_name="c", subcore_axis_name="s")

    @functools.partial(
        pl.kernel,
        out_type=jax.ShapeDtypeStruct((_SC_CORES, N, D), jnp.float32),
        mesh=mesh,
        scratch_types=[
            pltpu.VMEM_SHARED((N, D), jnp.float32),
            pltpu.VMEM((CW,), jnp.int32),
            pltpu.VMEM((CW, D), jnp.float32),
        ],
    )
    def k(m_hbm, i_hbm, z_hbm, o_hbm, acc_shared, idx_v, m_v):
        c = lax.axis_index("c")
        s = lax.axis_index("s")
        row0 = s * rows_per_sub
        pltpu.sync_copy(z_hbm.at[pl.ds(row0, rows_per_sub)],
                        acc_shared.at[pl.ds(row0, rows_per_sub)])
        plsc.subcore_barrier()
        base_chunk = c * ch_per_core + s * ch_per_worker

        @pl.loop(0, ch_per_worker)
        def _(j):
            ch = base_chunk + j
            pltpu.sync_copy(i_hbm.at[ch], idx_v)
            pltpu.sync_copy(m_hbm.at[pl.ds(ch * CW, CW)], m_v)
            pltpu.sync_copy(m_v, acc_shared.at[idx_v], add=True)

        plsc.subcore_barrier()
        pltpu.sync_copy(acc_shared.at[pl.ds(row0, rows_per_sub)],
                        o_hbm.at[c, pl.ds(row0, rows_per_sub)])

    return k(m2, idx2d, zeros_nd)


def kernel(node_features, edge_features, from_idx, to_idx,
           enc_node_W, enc_node_b, enc_edge_W, enc_edge_b,
           msg_W1, msg_b1, msg_W2, msg_b2,
           rmsg_W1, rmsg_b1, rmsg_W2, rmsg_b2,
           node_W, node_b, fc1_W, fc1_b, fc2_W, fc2_b):
    f32 = jnp.float32
    from_i = from_idx.astype(jnp.int32)
    to_i = to_idx.astype(jnp.int32)

    wf = jnp.concatenate([msg_W1[:D], rmsg_W1[D:2 * D]], axis=1)
    wt = jnp.concatenate([msg_W1[D:2 * D], rmsg_W1[:D]], axis=1)
    wcc = jnp.concatenate([msg_W1[2 * D:], rmsg_W1[2 * D:]], axis=1)
    bcc = jnp.concatenate([msg_b1, rmsg_b1]).reshape(1, 2 * H)
    nwa = node_W[:D]
    nwb = node_W[D:]
    benc = enc_node_b.reshape(1, D)
    bee = enc_edge_b.reshape(1, DE)
    b2 = msg_b2.reshape(1, H)
    rb2 = rmsg_b2.reshape(1, H)
    nb = node_b.reshape(1, D)
    fb1 = fc1_b.reshape(1, TD)
    fb2 = fc2_b.reshape(1, TD)

    gat_idx = jnp.concatenate([from_i, to_i + N]).reshape(1, E2)
    sct_idx = jnp.concatenate([to_i, from_i]).reshape(E2 // CW, CW)
    zeros_nd = jnp.zeros((N, D), f32)

    h, p = _encode_and_project(node_features, enc_node_W, benc, wf, wt)
    ec = _edge_term(edge_features, enc_edge_W, bee, wcc, bcc)

    for layer in range(NLAYERS):
        g = _sc_gather(p.reshape(2 * N, 2 * H), gat_idx)
        m = _messages(g.reshape(2, E, 2 * H), ec, msg_W2, b2, rmsg_W2, rb2)
        parts = _sc_scatter_add(m.reshape(E2, H), sct_idx, zeros_nd)
        if layer < NLAYERS - 1:
            h, p = _update_and_project(h, parts, nwa, nwb, nb, wf, wt)
        else:
            h = _update_final(h, parts, nwa, nwb, nb)

    t = _transform(h, fc1_W, fb1, fc2_W, fb2)
    x3 = h.reshape(NP, 2 * MS, D)
    t3 = t.reshape(NP, 2 * MS, TD)
    scores = _sinkhorn_scores(x3, t3)
    return scores.reshape(NP)


# speedup vs baseline: 2.8819x; 1.1057x over previous
import functools

import jax
import jax.numpy as jnp
from jax import lax
from jax.experimental import pallas as pl
from jax.experimental.pallas import tpu as pltpu
from jax.experimental.pallas import tpu_sc as plsc

N = 7680
E = 30720
D = 128
DE = 16
H = 128
TD = 64
MS = 15
NP = 256
NLAYERS = 3
TEMP = 0.1
SINK_ITERS = 20

E2 = 2 * E
NODE_BLK = 512
EDGE_BLK = 1024
GW = 128
CW = 128
PB = 16

_SC_CORES = 2
_SC_SUBCORES = 16
_SC_WORKERS = _SC_CORES * _SC_SUBCORES


def _enc_body(nf, wenc, benc, h_out):
    h_out[...] = jnp.dot(nf[...], wenc[...], preferred_element_type=jnp.float32) + benc[...]


def _encode(nf, wenc, benc):
    grid = (N // NODE_BLK,)
    return pl.pallas_call(
        _enc_body,
        grid=grid,
        in_specs=[
            pl.BlockSpec((NODE_BLK, D), lambda i: (i, 0)),
            pl.BlockSpec((D, D), lambda i: (0, 0)),
            pl.BlockSpec((1, D), lambda i: (0, 0)),
        ],
        out_specs=pl.BlockSpec((NODE_BLK, D), lambda i: (i, 0)),
        out_shape=jax.ShapeDtypeStruct((N, D), jnp.float32),
    )(nf, wenc, benc)


def _edge_term_body(ef, wee, bee, wcc, bcc, ec_out):
    t = jnp.dot(ef[...], wee[...], preferred_element_type=jnp.float32) + bee[...]
    ec_out[...] = jnp.dot(t, wcc[...], preferred_element_type=jnp.float32) + bcc[...]


def _edge_term(ef, wee, bee, wcc, bcc):
    grid = (E // EDGE_BLK,)
    return pl.pallas_call(
        _edge_term_body,
        grid=grid,
        in_specs=[
            pl.BlockSpec((EDGE_BLK, DE), lambda i: (i, 0)),
            pl.BlockSpec((DE, DE), lambda i: (0, 0)),
            pl.BlockSpec((1, DE), lambda i: (0, 0)),
            pl.BlockSpec((DE, 2 * H), lambda i: (0, 0)),
            pl.BlockSpec((1, 2 * H), lambda i: (0, 0)),
        ],
        out_specs=pl.BlockSpec((EDGE_BLK, 2 * H), lambda i: (i, 0)),
        out_shape=jax.ShapeDtypeStruct((E, 2 * H), jnp.float32),
    )(ef, wee, bee, wcc, bcc)


def _msg_body(g, ec, wf, wt, w2, b2, rw2, rb2, m_out):
    u = (jnp.dot(g[0], wf[...], preferred_element_type=jnp.float32)
         + jnp.dot(g[1], wt[...], preferred_element_type=jnp.float32)
         + ec[...])
    x = jnp.maximum(u, 0.0)
    m_out[0] = jnp.dot(x[:, :H], w2[...], preferred_element_type=jnp.float32) + b2[...]
    m_out[1] = jnp.dot(x[:, H:], rw2[...], preferred_element_type=jnp.float32) + rb2[...]


def _messages(g3, ec, wf, wt, w2, b2, rw2, rb2):
    grid = (E // EDGE_BLK,)
    return pl.pallas_call(
        _msg_body,
        grid=grid,
        in_specs=[
            pl.BlockSpec((2, EDGE_BLK, D), lambda i: (0, i, 0)),
            pl.BlockSpec((EDGE_BLK, 2 * H), lambda i: (i, 0)),
            pl.BlockSpec((D, 2 * H), lambda i: (0, 0)),
            pl.BlockSpec((D, 2 * H), lambda i: (0, 0)),
            pl.BlockSpec((H, H), lambda i: (0, 0)),
            pl.BlockSpec((1, H), lambda i: (0, 0)),
            pl.BlockSpec((H, H), lambda i: (0, 0)),
            pl.BlockSpec((1, H), lambda i: (0, 0)),
        ],
        out_specs=pl.BlockSpec((2, EDGE_BLK, H), lambda i: (0, i, 0)),
        out_shape=jax.ShapeDtypeStruct((2, E, H), jnp.float32),
    )(g3, ec, wf, wt, w2, b2, rw2, rb2)


def _update_final_body(h, p, nwa, nwb, nb, h_out):
    agg = p[0] + p[1]
    h_out[...] = (jnp.dot(h[...], nwa[...], preferred_element_type=jnp.float32)
                  + jnp.dot(agg, nwb[...], preferred_element_type=jnp.float32)
                  + nb[...])


def _update_final(h, parts, nwa, nwb, nb):
    grid = (N // NODE_BLK,)
    return pl.pallas_call(
        _update_final_body,
        grid=grid,
        in_specs=[
            pl.BlockSpec((NODE_BLK, D), lambda i: (i, 0)),
            pl.BlockSpec((2, NODE_BLK, H), lambda i: (0, i, 0)),
            pl.BlockSpec((D, D), lambda i: (0, 0)),
            pl.BlockSpec((H, D), lambda i: (0, 0)),
            pl.BlockSpec((1, D), lambda i: (0, 0)),
        ],
        out_specs=pl.BlockSpec((NODE_BLK, D), lambda i: (i, 0)),
        out_shape=jax.ShapeDtypeStruct((N, D), jnp.float32),
    )(h, parts, nwa, nwb, nb)


def _transform_body(h, w1, b1, w2, b2, t_out):
    t = jnp.maximum(jnp.dot(h[...], w1[...], preferred_element_type=jnp.float32) + b1[...], 0.0)
    t_out[...] = jnp.dot(t, w2[...], preferred_element_type=jnp.float32) + b2[...]


def _transform(h, w1, b1, w2, b2):
    grid = (N // NODE_BLK,)
    return pl.pallas_call(
        _transform_body,
        grid=grid,
        in_specs=[
            pl.BlockSpec((NODE_BLK, D), lambda i: (i, 0)),
            pl.BlockSpec((D, TD), lambda i: (0, 0)),
            pl.BlockSpec((1, TD), lambda i: (0, 0)),
            pl.BlockSpec((TD, TD), lambda i: (0, 0)),
            pl.BlockSpec((1, TD), lambda i: (0, 0)),
        ],
        out_specs=pl.BlockSpec((NODE_BLK, TD), lambda i: (i, 0)),
        out_shape=jax.ShapeDtypeStruct((N, TD), jnp.float32),
    )(h, w1, b1, w2, b2)


def _sinkhorn_body(x, t, out, s_ref, pc_ref):
    xb = x[...]
    tb = t[...]
    hq = xb[:, :MS, :]
    hc = xb[:, MS:, :]
    tq = tb[:, :MS, :]
    tct = jnp.swapaxes(tb[:, MS:, :], 1, 2)
    for b in range(PB):
        s_ref[b] = jnp.dot(tq[b], tct[b], preferred_element_type=jnp.float32)
    la = s_ref[...] * (1.0 / TEMP)

    def _iter(_, la):
        m2 = jnp.max(la, axis=2, keepdims=True)
        la = la - (m2 + jnp.log(jnp.sum(jnp.exp(la - m2), axis=2, keepdims=True)))
        m1 = jnp.max(la, axis=1, keepdims=True)
        la = la - (m1 + jnp.log(jnp.sum(jnp.exp(la - m1), axis=1, keepdims=True)))
        return la

    la = lax.fori_loop(0, SINK_ITERS, _iter, la)
    plan = jnp.exp(la)
    for b in range(PB):
        pc_ref[b] = jnp.dot(plan[b], hc[b], preferred_element_type=jnp.float32)
    diff = jnp.maximum(hq - pc_ref[...], 0.0)
    r = jnp.sum(jnp.sum(diff, axis=2), axis=1)
    out[...] = (-r).reshape(1, 1, PB)


def _sinkhorn_scores(x3, t3):
    grid = (NP // PB,)
    return pl.pallas_call(
        _sinkhorn_body,
        grid=grid,
        in_specs=[
            pl.BlockSpec((PB, 2 * MS, D), lambda i: (i, 0, 0)),
            pl.BlockSpec((PB, 2 * MS, TD), lambda i: (i, 0, 0)),
        ],
        out_specs=pl.BlockSpec((1, 1, PB), lambda i: (i, 0, 0)),
        out_shape=jax.ShapeDtypeStruct((NP // PB, 1, PB), jnp.float32),
        scratch_shapes=[
            pltpu.VMEM((PB, MS, MS), jnp.float32),
            pltpu.VMEM((PB, MS, D), jnp.float32),
        ],
    )(x3, t3)


def _sc_gather(table, idx2d):
    ni = idx2d.shape[1]
    cols = table.shape[1]
    mesh = plsc.VectorSubcoreMesh(core_axis_name="c", subcore_axis_name="s")

    @functools.partial(
        pl.kernel,
        out_type=jax.ShapeDtypeStruct((ni, cols), jnp.float32),
        mesh=mesh,
    )
    def k(tab_hbm, i_hbm, o_hbm):
        def body(i_vmem, o_vmem):
            pltpu.sync_copy(tab_hbm.at[i_vmem.at[0]], o_vmem)

        pltpu.emit_pipeline(
            body,
            grid=(ni // GW,),
            in_specs=[pl.BlockSpec((1, GW), lambda i: (0, i))],
            out_specs=[pl.BlockSpec((GW, cols), lambda i: (i, 0))],
            core_axis_name=("c", "s"),
            dimension_semantics=(pltpu.PARALLEL,),
        )(i_hbm, o_hbm)

    return k(table, idx2d)


def _sc_scatter_add(m2, idx2d, zeros_nd):
    nch = idx2d.shape[0]
    ch_per_core = nch // _SC_CORES
    ch_per_worker = nch // _SC_WORKERS
    rows_per_sub = N // _SC_SUBCORES
    mesh = plsc.VectorSubcoreMesh(core_axis_name="c", subcore_axis_name="s")

    @functools.partial(
        pl.kernel,
        out_type=jax.ShapeDtypeStruct((_SC_CORES, N, D), jnp.float32),
        mesh=mesh,
        scratch_types=[
            pltpu.VMEM_SHARED((N, D), jnp.float32),
            pltpu.VMEM((CW,), jnp.int32),
            pltpu.VMEM((CW, D), jnp.float32),
        ],
    )
    def k(m_hbm, i_hbm, z_hbm, o_hbm, acc_shared, idx_v, m_v):
        c = lax.axis_index("c")
        s = lax.axis_index("s")
        row0 = s * rows_per_sub
        pltpu.sync_copy(z_hbm.at[pl.ds(row0, rows_per_sub)],
                        acc_shared.at[pl.ds(row0, rows_per_sub)])
        plsc.subcore_barrier()
        base_chunk = c * ch_per_core + s * ch_per_worker

        @pl.loop(0, ch_per_worker)
        def _(j):
            ch = base_chunk + j
            pltpu.sync_copy(i_hbm.at[ch], idx_v)
            pltpu.sync_copy(m_hbm.at[pl.ds(ch * CW, CW)], m_v)
            pltpu.sync_copy(m_v, acc_shared.at[idx_v], add=True)

        plsc.subcore_barrier()
        pltpu.sync_copy(acc_shared.at[pl.ds(row0, rows_per_sub)],
                        o_hbm.at[c, pl.ds(row0, rows_per_sub)])

    return k(m2, idx2d, zeros_nd)


def kernel(node_features, edge_features, from_idx, to_idx,
           enc_node_W, enc_node_b, enc_edge_W, enc_edge_b,
           msg_W1, msg_b1, msg_W2, msg_b2,
           rmsg_W1, rmsg_b1, rmsg_W2, rmsg_b2,
           node_W, node_b, fc1_W, fc1_b, fc2_W, fc2_b):
    f32 = jnp.float32
    from_i = from_idx.astype(jnp.int32)
    to_i = to_idx.astype(jnp.int32)

    wf = jnp.concatenate([msg_W1[:D], rmsg_W1[D:2 * D]], axis=1)
    wt = jnp.concatenate([msg_W1[D:2 * D], rmsg_W1[:D]], axis=1)
    wcc = jnp.concatenate([msg_W1[2 * D:], rmsg_W1[2 * D:]], axis=1)
    bcc = jnp.concatenate([msg_b1, rmsg_b1]).reshape(1, 2 * H)
    nwa = node_W[:D]
    nwb = node_W[D:]
    benc = enc_node_b.reshape(1, D)
    bee = enc_edge_b.reshape(1, DE)
    b2 = msg_b2.reshape(1, H)
    rb2 = rmsg_b2.reshape(1, H)
    nb = node_b.reshape(1, D)
    fb1 = fc1_b.reshape(1, TD)
    fb2 = fc2_b.reshape(1, TD)

    gat_idx = jnp.concatenate([from_i, to_i]).reshape(1, E2)
    sct_idx = jnp.concatenate([to_i, from_i]).reshape(E2 // CW, CW)
    zeros_nd = jnp.zeros((N, D), f32)

    h = _encode(node_features, enc_node_W, benc)
    ec = _edge_term(edge_features, enc_edge_W, bee, wcc, bcc)

    for _ in range(NLAYERS):
        g = _sc_gather(h, gat_idx)
        m = _messages(g.reshape(2, E, D), ec, wf, wt, msg_W2, b2, rmsg_W2, rb2)
        parts = _sc_scatter_add(m.reshape(E2, H), sct_idx, zeros_nd)
        h = _update_final(h, parts, nwa, nwb, nb)

    t = _transform(h, fc1_W, fb1, fc2_W, fb2)
    x3 = h.reshape(NP, 2 * MS, D)
    t3 = t.reshape(NP, 2 * MS, TD)
    scores = _sinkhorn_scores(x3, t3)
    return scores.reshape(NP)


# speedup vs baseline: 3.1828x; 1.1044x over previous
import functools

import jax
import jax.numpy as jnp
from jax import lax
from jax.experimental import pallas as pl
from jax.experimental.pallas import tpu as pltpu
from jax.experimental.pallas import tpu_sc as plsc

N = 7680
E = 30720
D = 128
DE = 16
H = 128
TD = 64
MS = 15
NP = 256
NLAYERS = 3
TEMP = 0.1
SINK_ITERS = 20

E2 = 2 * E
NODE_BLK = 512
EDGE_BLK = 1024
GW = 128
CW = 128
PB = 16

_SC_CORES = 2
_SC_SUBCORES = 16
_SC_WORKERS = _SC_CORES * _SC_SUBCORES


def _enc_body(nf, wenc, benc, h_out):
    h_out[...] = jnp.dot(nf[...], wenc[...], preferred_element_type=jnp.float32) + benc[...]


def _encode(nf, wenc, benc):
    grid = (N // NODE_BLK,)
    return pl.pallas_call(
        _enc_body,
        grid=grid,
        in_specs=[
            pl.BlockSpec((NODE_BLK, D), lambda i: (i, 0)),
            pl.BlockSpec((D, D), lambda i: (0, 0)),
            pl.BlockSpec((1, D), lambda i: (0, 0)),
        ],
        out_specs=pl.BlockSpec((NODE_BLK, D), lambda i: (i, 0)),
        out_shape=jax.ShapeDtypeStruct((N, D), jnp.float32),
    )(nf, wenc, benc)


def _msg_body(g, ef, wee, bee, wcc, bcc, wf, wt, w2, b2, rw2, rb2, m_out):
    e = jnp.dot(ef[...], wee[...], preferred_element_type=jnp.float32) + bee[...]
    ec = jnp.dot(e, wcc[...], preferred_element_type=jnp.float32) + bcc[...]
    u = (jnp.dot(g[0], wf[...], preferred_element_type=jnp.float32)
         + jnp.dot(g[1], wt[...], preferred_element_type=jnp.float32)
         + ec)
    x = jnp.maximum(u, 0.0)
    m_out[0] = jnp.dot(x[:, :H], w2[...], preferred_element_type=jnp.float32) + b2[...]
    m_out[1] = jnp.dot(x[:, H:], rw2[...], preferred_element_type=jnp.float32) + rb2[...]


def _messages(g3, ef, wee, bee, wcc, bcc, wf, wt, w2, b2, rw2, rb2):
    grid = (E // EDGE_BLK,)
    return pl.pallas_call(
        _msg_body,
        grid=grid,
        in_specs=[
            pl.BlockSpec((2, EDGE_BLK, D), lambda i: (0, i, 0)),
            pl.BlockSpec((EDGE_BLK, DE), lambda i: (i, 0)),
            pl.BlockSpec((DE, DE), lambda i: (0, 0)),
            pl.BlockSpec((1, DE), lambda i: (0, 0)),
            pl.BlockSpec((DE, 2 * H), lambda i: (0, 0)),
            pl.BlockSpec((1, 2 * H), lambda i: (0, 0)),
            pl.BlockSpec((D, 2 * H), lambda i: (0, 0)),
            pl.BlockSpec((D, 2 * H), lambda i: (0, 0)),
            pl.BlockSpec((H, H), lambda i: (0, 0)),
            pl.BlockSpec((1, H), lambda i: (0, 0)),
            pl.BlockSpec((H, H), lambda i: (0, 0)),
            pl.BlockSpec((1, H), lambda i: (0, 0)),
        ],
        out_specs=pl.BlockSpec((2, EDGE_BLK, H), lambda i: (0, i, 0)),
        out_shape=jax.ShapeDtypeStruct((2, E, H), jnp.float32),
    )(g3, ef, wee, bee, wcc, bcc, wf, wt, w2, b2, rw2, rb2)


def _update_final_body(h, p, nwa, nwb, nb, h_out):
    agg = p[0] + p[1]
    h_out[...] = (jnp.dot(h[...], nwa[...], preferred_element_type=jnp.float32)
                  + jnp.dot(agg, nwb[...], preferred_element_type=jnp.float32)
                  + nb[...])


def _update_final(h, parts, nwa, nwb, nb):
    grid = (N // NODE_BLK,)
    return pl.pallas_call(
        _update_final_body,
        grid=grid,
        in_specs=[
            pl.BlockSpec((NODE_BLK, D), lambda i: (i, 0)),
            pl.BlockSpec((2, NODE_BLK, H), lambda i: (0, i, 0)),
            pl.BlockSpec((D, D), lambda i: (0, 0)),
            pl.BlockSpec((H, D), lambda i: (0, 0)),
            pl.BlockSpec((1, D), lambda i: (0, 0)),
        ],
        out_specs=pl.BlockSpec((NODE_BLK, D), lambda i: (i, 0)),
        out_shape=jax.ShapeDtypeStruct((N, D), jnp.float32),
    )(h, parts, nwa, nwb, nb)


def _transform_body(h, w1, b1, w2, b2, t_out):
    t = jnp.maximum(jnp.dot(h[...], w1[...], preferred_element_type=jnp.float32) + b1[...], 0.0)
    t_out[...] = jnp.dot(t, w2[...], preferred_element_type=jnp.float32) + b2[...]


def _transform(h, w1, b1, w2, b2):
    grid = (N // NODE_BLK,)
    return pl.pallas_call(
        _transform_body,
        grid=grid,
        in_specs=[
            pl.BlockSpec((NODE_BLK, D), lambda i: (i, 0)),
            pl.BlockSpec((D, TD), lambda i: (0, 0)),
            pl.BlockSpec((1, TD), lambda i: (0, 0)),
            pl.BlockSpec((TD, TD), lambda i: (0, 0)),
            pl.BlockSpec((1, TD), lambda i: (0, 0)),
        ],
        out_specs=pl.BlockSpec((NODE_BLK, TD), lambda i: (i, 0)),
        out_shape=jax.ShapeDtypeStruct((N, TD), jnp.float32),
    )(h, w1, b1, w2, b2)


def _sinkhorn_body(x, t, out, s_ref, pc_ref):
    xb = x[...]
    tb = t[...]
    hq = xb[:, :MS, :]
    hc = xb[:, MS:, :]
    tq = tb[:, :MS, :]
    tct = jnp.swapaxes(tb[:, MS:, :], 1, 2)
    for b in range(PB):
        s_ref[b] = jnp.dot(tq[b], tct[b], preferred_element_type=jnp.float32)
    s_pm = s_ref[...]

    eye_pb = jnp.eye(PB, dtype=jnp.float32)
    eye_ms = jnp.eye(MS, dtype=jnp.float32)
    la3 = jnp.stack(
        [lax.dot_general(s_pm[:, i, :], eye_pb, (((0,), (0,)), ((), ())),
                         preferred_element_type=jnp.float32)
         for i in range(MS)], axis=0) * (1.0 / TEMP)

    def _iter(_, la):
        m2 = jnp.max(la, axis=1, keepdims=True)
        la = la - (m2 + jnp.log(jnp.sum(jnp.exp(la - m2), axis=1, keepdims=True)))
        m1 = jnp.max(la, axis=0, keepdims=True)
        la = la - (m1 + jnp.log(jnp.sum(jnp.exp(la - m1), axis=0, keepdims=True)))
        return la

    la3 = lax.fori_loop(0, SINK_ITERS, _iter, la3)
    plan3 = jnp.exp(la3)
    plan_pm = jnp.stack(
        [lax.dot_general(plan3[i], eye_ms, (((0,), (0,)), ((), ())),
                         preferred_element_type=jnp.float32)
         for i in range(MS)], axis=1)
    for b in range(PB):
        pc_ref[b] = jnp.dot(plan_pm[b], hc[b], preferred_element_type=jnp.float32)
    diff = jnp.maximum(hq - pc_ref[...], 0.0)
    r = jnp.sum(jnp.sum(diff, axis=2), axis=1)
    out[...] = (-r).reshape(1, 1, PB)


def _sinkhorn_scores(x3, t3):
    grid = (NP // PB,)
    return pl.pallas_call(
        _sinkhorn_body,
        grid=grid,
        in_specs=[
            pl.BlockSpec((PB, 2 * MS, D), lambda i: (i, 0, 0)),
            pl.BlockSpec((PB, 2 * MS, TD), lambda i: (i, 0, 0)),
        ],
        out_specs=pl.BlockSpec((1, 1, PB), lambda i: (i, 0, 0)),
        out_shape=jax.ShapeDtypeStruct((NP // PB, 1, PB), jnp.float32),
        scratch_shapes=[
            pltpu.VMEM((PB, MS, MS), jnp.float32),
            pltpu.VMEM((PB, MS, D), jnp.float32),
        ],
    )(x3, t3)


def _sc_gather(table, idx2d):
    ni = idx2d.shape[1]
    cols = table.shape[1]
    mesh = plsc.VectorSubcoreMesh(core_axis_name="c", subcore_axis_name="s")

    @functools.partial(
        pl.kernel,
        out_type=jax.ShapeDtypeStruct((ni, cols), jnp.float32),
        mesh=mesh,
    )
    def k(tab_hbm, i_hbm, o_hbm):
        def body(i_vmem, o_vmem):
            pltpu.sync_copy(tab_hbm.at[i_vmem.at[0]], o_vmem)

        pltpu.emit_pipeline(
            body,
            grid=(ni // GW,),
            in_specs=[pl.BlockSpec((1, GW), lambda i: (0, i))],
            out_specs=[pl.BlockSpec((GW, cols), lambda i: (i, 0))],
            core_axis_name=("c", "s"),
            dimension_semantics=(pltpu.PARALLEL,),
        )(i_hbm, o_hbm)

    return k(table, idx2d)


def _sc_scatter_add(m2, idx2d, zeros_nd):
    nch = idx2d.shape[0]
    ch_per_core = nch // _SC_CORES
    ch_per_worker = nch // _SC_WORKERS
    rows_per_sub = N // _SC_SUBCORES
    mesh = plsc.VectorSubcoreMesh(core_axis_name="c", subcore_axis_name="s")

    @functools.partial(
        pl.kernel,
        out_type=jax.ShapeDtypeStruct((_SC_CORES, N, D), jnp.float32),
        mesh=mesh,
        scratch_types=[
            pltpu.VMEM_SHARED((N, D), jnp.float32),
            pltpu.VMEM((CW,), jnp.int32),
            pltpu.VMEM((CW, D), jnp.float32),
        ],
    )
    def k(m_hbm, i_hbm, z_hbm, o_hbm, acc_shared, idx_v, m_v):
        c = lax.axis_index("c")
        s = lax.axis_index("s")
        row0 = s * rows_per_sub
        pltpu.sync_copy(z_hbm.at[pl.ds(row0, rows_per_sub)],
                        acc_shared.at[pl.ds(row0, rows_per_sub)])
        plsc.subcore_barrier()
        base_chunk = c * ch_per_core + s * ch_per_worker

        @pl.loop(0, ch_per_worker)
        def _(j):
            ch = base_chunk + j
            pltpu.sync_copy(i_hbm.at[ch], idx_v)
            pltpu.sync_copy(m_hbm.at[pl.ds(ch * CW, CW)], m_v)
            pltpu.sync_copy(m_v, acc_shared.at[idx_v], add=True)

        plsc.subcore_barrier()
        pltpu.sync_copy(acc_shared.at[pl.ds(row0, rows_per_sub)],
                        o_hbm.at[c, pl.ds(row0, rows_per_sub)])

    return k(m2, idx2d, zeros_nd)


def kernel(node_features, edge_features, from_idx, to_idx,
           enc_node_W, enc_node_b, enc_edge_W, enc_edge_b,
           msg_W1, msg_b1, msg_W2, msg_b2,
           rmsg_W1, rmsg_b1, rmsg_W2, rmsg_b2,
           node_W, node_b, fc1_W, fc1_b, fc2_W, fc2_b):
    f32 = jnp.float32
    from_i = from_idx.astype(jnp.int32)
    to_i = to_idx.astype(jnp.int32)

    wf = jnp.concatenate([msg_W1[:D], rmsg_W1[D:2 * D]], axis=1)
    wt = jnp.concatenate([msg_W1[D:2 * D], rmsg_W1[:D]], axis=1)
    wcc = jnp.concatenate([msg_W1[2 * D:], rmsg_W1[2 * D:]], axis=1)
    bcc = jnp.concatenate([msg_b1, rmsg_b1]).reshape(1, 2 * H)
    nwa = node_W[:D]
    nwb = node_W[D:]
    benc = enc_node_b.reshape(1, D)
    bee = enc_edge_b.reshape(1, DE)
    b2 = msg_b2.reshape(1, H)
    rb2 = rmsg_b2.reshape(1, H)
    nb = node_b.reshape(1, D)
    fb1 = fc1_b.reshape(1, TD)
    fb2 = fc2_b.reshape(1, TD)

    gat_idx = jnp.concatenate([from_i, to_i]).reshape(1, E2)
    sct_idx = jnp.concatenate([to_i, from_i]).reshape(E2 // CW, CW)
    zeros_nd = jnp.zeros((N, D), f32)

    h = _encode(node_features, enc_node_W, benc)

    for _ in range(NLAYERS):
        g = _sc_gather(h, gat_idx)
        m = _messages(g.reshape(2, E, D), edge_features, enc_edge_W, bee,
                      wcc, bcc, wf, wt, msg_W2, b2, rmsg_W2, rb2)
        parts = _sc_scatter_add(m.reshape(E2, H), sct_idx, zeros_nd)
        h = _update_final(h, parts, nwa, nwb, nb)

    t = _transform(h, fc1_W, fb1, fc2_W, fb2)
    x3 = h.reshape(NP, 2 * MS, D)
    t3 = t.reshape(NP, 2 * MS, TD)
    scores = _sinkhorn_scores(x3, t3)
    return scores.reshape(NP)


# speedup vs baseline: 3.4651x; 1.0887x over previous
import functools

import jax
import jax.numpy as jnp
from jax import lax
from jax.experimental import pallas as pl
from jax.experimental.pallas import tpu as pltpu
from jax.experimental.pallas import tpu_sc as plsc

N = 7680
E = 30720
D = 128
DE = 16
H = 128
TD = 64
MS = 15
NP = 256
NLAYERS = 3
TEMP = 0.1
SINK_ITERS = 20

E2 = 2 * E
NODE_BLK = 512
EDGE_BLK = 1024
GW = 128
CW = 128
PB = 128

_SC_CORES = 2
_SC_SUBCORES = 16
_SC_WORKERS = _SC_CORES * _SC_SUBCORES


def _enc_body(nf, wenc, benc, h_out):
    h_out[...] = jnp.dot(nf[...], wenc[...], preferred_element_type=jnp.float32) + benc[...]


def _encode(nf, wenc, benc):
    grid = (N // NODE_BLK,)
    return pl.pallas_call(
        _enc_body,
        grid=grid,
        in_specs=[
            pl.BlockSpec((NODE_BLK, D), lambda i: (i, 0)),
            pl.BlockSpec((D, D), lambda i: (0, 0)),
            pl.BlockSpec((1, D), lambda i: (0, 0)),
        ],
        out_specs=pl.BlockSpec((NODE_BLK, D), lambda i: (i, 0)),
        out_shape=jax.ShapeDtypeStruct((N, D), jnp.float32),
    )(nf, wenc, benc)


def _msg_body(g, ef, wee, bee, wcc, bcc, wf, wt, w2, b2, rw2, rb2, m_out):
    e = jnp.dot(ef[...], wee[...], preferred_element_type=jnp.float32) + bee[...]
    ec = jnp.dot(e, wcc[...], preferred_element_type=jnp.float32) + bcc[...]
    u = (jnp.dot(g[0], wf[...], preferred_element_type=jnp.float32)
         + jnp.dot(g[1], wt[...], preferred_element_type=jnp.float32)
         + ec)
    x = jnp.maximum(u, 0.0)
    m_out[0] = jnp.dot(x[:, :H], w2[...], preferred_element_type=jnp.float32) + b2[...]
    m_out[1] = jnp.dot(x[:, H:], rw2[...], preferred_element_type=jnp.float32) + rb2[...]


def _messages(g3, ef, wee, bee, wcc, bcc, wf, wt, w2, b2, rw2, rb2):
    grid = (E // EDGE_BLK,)
    return pl.pallas_call(
        _msg_body,
        grid=grid,
        in_specs=[
            pl.BlockSpec((2, EDGE_BLK, D), lambda i: (0, i, 0)),
            pl.BlockSpec((EDGE_BLK, DE), lambda i: (i, 0)),
            pl.BlockSpec((DE, DE), lambda i: (0, 0)),
            pl.BlockSpec((1, DE), lambda i: (0, 0)),
            pl.BlockSpec((DE, 2 * H), lambda i: (0, 0)),
            pl.BlockSpec((1, 2 * H), lambda i: (0, 0)),
            pl.BlockSpec((D, 2 * H), lambda i: (0, 0)),
            pl.BlockSpec((D, 2 * H), lambda i: (0, 0)),
            pl.BlockSpec((H, H), lambda i: (0, 0)),
            pl.BlockSpec((1, H), lambda i: (0, 0)),
            pl.BlockSpec((H, H), lambda i: (0, 0)),
            pl.BlockSpec((1, H), lambda i: (0, 0)),
        ],
        out_specs=pl.BlockSpec((2, EDGE_BLK, H), lambda i: (0, i, 0)),
        out_shape=jax.ShapeDtypeStruct((2, E, H), jnp.float32),
    )(g3, ef, wee, bee, wcc, bcc, wf, wt, w2, b2, rw2, rb2)


def _update_final_body(h, p, nwa, nwb, nb, h_out):
    agg = p[0] + p[1]
    h_out[...] = (jnp.dot(h[...], nwa[...], preferred_element_type=jnp.float32)
                  + jnp.dot(agg, nwb[...], preferred_element_type=jnp.float32)
                  + nb[...])


def _update_final(h, parts, nwa, nwb, nb):
    grid = (N // NODE_BLK,)
    return pl.pallas_call(
        _update_final_body,
        grid=grid,
        in_specs=[
            pl.BlockSpec((NODE_BLK, D), lambda i: (i, 0)),
            pl.BlockSpec((2, NODE_BLK, H), lambda i: (0, i, 0)),
            pl.BlockSpec((D, D), lambda i: (0, 0)),
            pl.BlockSpec((H, D), lambda i: (0, 0)),
            pl.BlockSpec((1, D), lambda i: (0, 0)),
        ],
        out_specs=pl.BlockSpec((NODE_BLK, D), lambda i: (i, 0)),
        out_shape=jax.ShapeDtypeStruct((N, D), jnp.float32),
    )(h, parts, nwa, nwb, nb)


def _transform_body(h, w1, b1, w2, b2, t_out):
    t = jnp.maximum(jnp.dot(h[...], w1[...], preferred_element_type=jnp.float32) + b1[...], 0.0)
    t_out[...] = jnp.dot(t, w2[...], preferred_element_type=jnp.float32) + b2[...]


def _transform(h, w1, b1, w2, b2):
    grid = (N // NODE_BLK,)
    return pl.pallas_call(
        _transform_body,
        grid=grid,
        in_specs=[
            pl.BlockSpec((NODE_BLK, D), lambda i: (i, 0)),
            pl.BlockSpec((D, TD), lambda i: (0, 0)),
            pl.BlockSpec((1, TD), lambda i: (0, 0)),
            pl.BlockSpec((TD, TD), lambda i: (0, 0)),
            pl.BlockSpec((1, TD), lambda i: (0, 0)),
        ],
        out_specs=pl.BlockSpec((NODE_BLK, TD), lambda i: (i, 0)),
        out_shape=jax.ShapeDtypeStruct((N, TD), jnp.float32),
    )(h, w1, b1, w2, b2)


def _sinkhorn_body(x, t, out, s_ref, pc_ref):
    xb = x[...]
    tb = t[...]
    hq = xb[:, :MS, :]
    hc = xb[:, MS:, :]
    tq = tb[:, :MS, :]
    tct = jnp.swapaxes(tb[:, MS:, :], 1, 2)
    for b in range(PB):
        s_ref[b] = jnp.dot(tq[b], tct[b], preferred_element_type=jnp.float32)
    s_pm = s_ref[...]

    eye_pb = jnp.eye(PB, dtype=jnp.float32)
    eye_ms = jnp.eye(MS, dtype=jnp.float32)
    la3 = jnp.stack(
        [lax.dot_general(s_pm[:, i, :], eye_pb, (((0,), (0,)), ((), ())),
                         preferred_element_type=jnp.float32)
         for i in range(MS)], axis=0) * (1.0 / TEMP)

    def _iter(_, la):
        m2 = jnp.max(la, axis=1, keepdims=True)
        la = la - (m2 + jnp.log(jnp.sum(jnp.exp(la - m2), axis=1, keepdims=True)))
        m1 = jnp.max(la, axis=0, keepdims=True)
        la = la - (m1 + jnp.log(jnp.sum(jnp.exp(la - m1), axis=0, keepdims=True)))
        return la

    la3 = lax.fori_loop(0, SINK_ITERS, _iter, la3)
    plan3 = jnp.exp(la3)
    plan_pm = jnp.stack(
        [lax.dot_general(plan3[i], eye_ms, (((0,), (0,)), ((), ())),
                         preferred_element_type=jnp.float32)
         for i in range(MS)], axis=1)
    for b in range(PB):
        pc_ref[b] = jnp.dot(plan_pm[b], hc[b], preferred_element_type=jnp.float32)
    diff = jnp.maximum(hq - pc_ref[...], 0.0)
    r = jnp.sum(jnp.sum(diff, axis=2), axis=1)
    out[...] = (-r).reshape(1, 1, PB)


def _sinkhorn_scores(x3, t3):
    grid = (NP // PB,)
    return pl.pallas_call(
        _sinkhorn_body,
        grid=grid,
        in_specs=[
            pl.BlockSpec((PB, 2 * MS, D), lambda i: (i, 0, 0)),
            pl.BlockSpec((PB, 2 * MS, TD), lambda i: (i, 0, 0)),
        ],
        out_specs=pl.BlockSpec((1, 1, PB), lambda i: (i, 0, 0)),
        out_shape=jax.ShapeDtypeStruct((NP // PB, 1, PB), jnp.float32),
        scratch_shapes=[
            pltpu.VMEM((PB, MS, MS), jnp.float32),
            pltpu.VMEM((PB, MS, D), jnp.float32),
        ],
    )(x3, t3)


def _sc_gather(table, idx2d):
    ni = idx2d.shape[1]
    cols = table.shape[1]
    mesh = plsc.VectorSubcoreMesh(core_axis_name="c", subcore_axis_name="s")

    @functools.partial(
        pl.kernel,
        out_type=jax.ShapeDtypeStruct((ni, cols), jnp.float32),
        mesh=mesh,
    )
    def k(tab_hbm, i_hbm, o_hbm):
        def body(i_vmem, o_vmem):
            pltpu.sync_copy(tab_hbm.at[i_vmem.at[0]], o_vmem)

        pltpu.emit_pipeline(
            body,
            grid=(ni // GW,),
            in_specs=[pl.BlockSpec((1, GW), lambda i: (0, i))],
            out_specs=[pl.BlockSpec((GW, cols), lambda i: (i, 0))],
            core_axis_name=("c", "s"),
            dimension_semantics=(pltpu.PARALLEL,),
        )(i_hbm, o_hbm)

    return k(table, idx2d)


def _sc_scatter_add(m2, idx2d, zeros_nd):
    nch = idx2d.shape[0]
    ch_per_core = nch // _SC_CORES
    ch_per_worker = nch // _SC_WORKERS
    rows_per_sub = N // _SC_SUBCORES
    mesh = plsc.VectorSubcoreMesh(core_axis_name="c", subcore_axis_name="s")

    @functools.partial(
        pl.kernel,
        out_type=jax.ShapeDtypeStruct((_SC_CORES, N, D), jnp.float32),
        mesh=mesh,
        scratch_types=[
            pltpu.VMEM_SHARED((N, D), jnp.float32),
            pltpu.VMEM((CW,), jnp.int32),
            pltpu.VMEM((CW, D), jnp.float32),
        ],
    )
    def k(m_hbm, i_hbm, z_hbm, o_hbm, acc_shared, idx_v, m_v):
        c = lax.axis_index("c")
        s = lax.axis_index("s")
        row0 = s * rows_per_sub
        pltpu.sync_copy(z_hbm.at[pl.ds(row0, rows_per_sub)],
                        acc_shared.at[pl.ds(row0, rows_per_sub)])
        plsc.subcore_barrier()
        base_chunk = c * ch_per_core + s * ch_per_worker

        @pl.loop(0, ch_per_worker)
        def _(j):
            ch = base_chunk + j
            pltpu.sync_copy(i_hbm.at[ch], idx_v)
            pltpu.sync_copy(m_hbm.at[pl.ds(ch * CW, CW)], m_v)
            pltpu.sync_copy(m_v, acc_shared.at[idx_v], add=True)

        plsc.subcore_barrier()
        pltpu.sync_copy(acc_shared.at[pl.ds(row0, rows_per_sub)],
                        o_hbm.at[c, pl.ds(row0, rows_per_sub)])

    return k(m2, idx2d, zeros_nd)


def kernel(node_features, edge_features, from_idx, to_idx,
           enc_node_W, enc_node_b, enc_edge_W, enc_edge_b,
           msg_W1, msg_b1, msg_W2, msg_b2,
           rmsg_W1, rmsg_b1, rmsg_W2, rmsg_b2,
           node_W, node_b, fc1_W, fc1_b, fc2_W, fc2_b):
    f32 = jnp.float32
    from_i = from_idx.astype(jnp.int32)
    to_i = to_idx.astype(jnp.int32)

    wf = jnp.concatenate([msg_W1[:D], rmsg_W1[D:2 * D]], axis=1)
    wt = jnp.concatenate([msg_W1[D:2 * D], rmsg_W1[:D]], axis=1)
    wcc = jnp.concatenate([msg_W1[2 * D:], rmsg_W1[2 * D:]], axis=1)
    bcc = jnp.concatenate([msg_b1, rmsg_b1]).reshape(1, 2 * H)
    nwa = node_W[:D]
    nwb = node_W[D:]
    benc = enc_node_b.reshape(1, D)
    bee = enc_edge_b.reshape(1, DE)
    b2 = msg_b2.reshape(1, H)
    rb2 = rmsg_b2.reshape(1, H)
    nb = node_b.reshape(1, D)
    fb1 = fc1_b.reshape(1, TD)
    fb2 = fc2_b.reshape(1, TD)

    gat_idx = jnp.concatenate([from_i, to_i]).reshape(1, E2)
    sct_idx = jnp.concatenate([to_i, from_i]).reshape(E2 // CW, CW)
    zeros_nd = jnp.zeros((N, D), f32)

    h = _encode(node_features, enc_node_W, benc)

    for _ in range(NLAYERS):
        g = _sc_gather(h, gat_idx)
        m = _messages(g.reshape(2, E, D), edge_features, enc_edge_W, bee,
                      wcc, bcc, wf, wt, msg_W2, b2, rmsg_W2, rb2)
        parts = _sc_scatter_add(m.reshape(E2, H), sct_idx, zeros_nd)
        h = _update_final(h, parts, nwa, nwb, nb)

    t = _transform(h, fc1_W, fb1, fc2_W, fb2)
    x3 = h.reshape(NP, 2 * MS, D)
    t3 = t.reshape(NP, 2 * MS, TD)
    scores = _sinkhorn_scores(x3, t3)
    return scores.reshape(NP)


# speedup vs baseline: 3.9379x; 1.1364x over previous
import functools

import jax
import jax.numpy as jnp
from jax import lax
from jax.experimental import pallas as pl
from jax.experimental.pallas import tpu as pltpu
from jax.experimental.pallas import tpu_sc as plsc

N = 7680
E = 30720
D = 128
DE = 16
H = 128
TD = 64
MS = 15
NP = 256
NLAYERS = 3
TEMP = 0.1
SINK_ITERS = 20

E2 = 2 * E
NODE_BLK = 512
EDGE_BLK = 2048
GW = 128
CW = 128
PB = 128

_SC_CORES = 2
_SC_SUBCORES = 16
_SC_WORKERS = _SC_CORES * _SC_SUBCORES


def _enc_body(nf, wenc, benc, h_out):
    h_out[...] = jnp.dot(nf[...], wenc[...], preferred_element_type=jnp.float32) + benc[...]


def _encode(nf, wenc, benc):
    grid = (N // NODE_BLK,)
    return pl.pallas_call(
        _enc_body,
        grid=grid,
        in_specs=[
            pl.BlockSpec((NODE_BLK, D), lambda i: (i, 0)),
            pl.BlockSpec((D, D), lambda i: (0, 0)),
            pl.BlockSpec((1, D), lambda i: (0, 0)),
        ],
        out_specs=pl.BlockSpec((NODE_BLK, D), lambda i: (i, 0)),
        out_shape=jax.ShapeDtypeStruct((N, D), jnp.float32),
    )(nf, wenc, benc)


def _msg_body(g, ef, wcomb, bcomb, wft, w2d, b2d, m_out):
    ec = jnp.dot(ef[...], wcomb[...], preferred_element_type=jnp.float32) + bcomb[...]
    gg = jnp.concatenate([g[0], g[1]], axis=1)
    u = jnp.dot(gg, wft[...], preferred_element_type=jnp.float32) + ec
    x = jnp.maximum(u, 0.0)
    m = jnp.dot(x, w2d[...], preferred_element_type=jnp.float32) + b2d[...]
    m_out[0] = m[:, :H]
    m_out[1] = m[:, H:]


def _messages(g3, ef, wcomb, bcomb, wft, w2d, b2d):
    grid = (E // EDGE_BLK,)
    return pl.pallas_call(
        _msg_body,
        grid=grid,
        in_specs=[
            pl.BlockSpec((2, EDGE_BLK, D), lambda i: (0, i, 0)),
            pl.BlockSpec((EDGE_BLK, DE), lambda i: (i, 0)),
            pl.BlockSpec((DE, 2 * H), lambda i: (0, 0)),
            pl.BlockSpec((1, 2 * H), lambda i: (0, 0)),
            pl.BlockSpec((2 * D, 2 * H), lambda i: (0, 0)),
            pl.BlockSpec((2 * H, 2 * H), lambda i: (0, 0)),
            pl.BlockSpec((1, 2 * H), lambda i: (0, 0)),
        ],
        out_specs=pl.BlockSpec((2, EDGE_BLK, H), lambda i: (0, i, 0)),
        out_shape=jax.ShapeDtypeStruct((2, E, H), jnp.float32),
    )(g3, ef, wcomb, bcomb, wft, w2d, b2d)


def _update_final_body(h, p, nwa, nwb, nb, h_out):
    agg = p[0] + p[1]
    h_out[...] = (jnp.dot(h[...], nwa[...], preferred_element_type=jnp.float32)
                  + jnp.dot(agg, nwb[...], preferred_element_type=jnp.float32)
                  + nb[...])


def _update_final(h, parts, nwa, nwb, nb):
    grid = (N // NODE_BLK,)
    return pl.pallas_call(
        _update_final_body,
        grid=grid,
        in_specs=[
            pl.BlockSpec((NODE_BLK, D), lambda i: (i, 0)),
            pl.BlockSpec((2, NODE_BLK, H), lambda i: (0, i, 0)),
            pl.BlockSpec((D, D), lambda i: (0, 0)),
            pl.BlockSpec((H, D), lambda i: (0, 0)),
            pl.BlockSpec((1, D), lambda i: (0, 0)),
        ],
        out_specs=pl.BlockSpec((NODE_BLK, D), lambda i: (i, 0)),
        out_shape=jax.ShapeDtypeStruct((N, D), jnp.float32),
    )(h, parts, nwa, nwb, nb)


def _transform_body(h, w1, b1, w2, b2, t_out):
    t = jnp.maximum(jnp.dot(h[...], w1[...], preferred_element_type=jnp.float32) + b1[...], 0.0)
    t_out[...] = jnp.dot(t, w2[...], preferred_element_type=jnp.float32) + b2[...]


def _transform(h, w1, b1, w2, b2):
    grid = (N // NODE_BLK,)
    return pl.pallas_call(
        _transform_body,
        grid=grid,
        in_specs=[
            pl.BlockSpec((NODE_BLK, D), lambda i: (i, 0)),
            pl.BlockSpec((D, TD), lambda i: (0, 0)),
            pl.BlockSpec((1, TD), lambda i: (0, 0)),
            pl.BlockSpec((TD, TD), lambda i: (0, 0)),
            pl.BlockSpec((1, TD), lambda i: (0, 0)),
        ],
        out_specs=pl.BlockSpec((NODE_BLK, TD), lambda i: (i, 0)),
        out_shape=jax.ShapeDtypeStruct((N, TD), jnp.float32),
    )(h, w1, b1, w2, b2)


def _sinkhorn_body(x, t, out, s_ref, pc_ref, hq_ref):
    xb = x[...]
    tb = t[...]
    for b in range(PB):
        tq_b = tb[30 * b:30 * b + MS, :]
        tc_b = tb[30 * b + MS:30 * b + 30, :]
        s_ref[b] = lax.dot_general(tq_b, tc_b, (((1,), (1,)), ((), ())),
                                   preferred_element_type=jnp.float32)
        hq_ref[b] = xb[30 * b:30 * b + MS, :]
    s_pm = s_ref[...]

    eye_pb = jnp.eye(PB, dtype=jnp.float32)
    eye_ms = jnp.eye(MS, dtype=jnp.float32)
    la3 = jnp.stack(
        [lax.dot_general(s_pm[:, i, :], eye_pb, (((0,), (0,)), ((), ())),
                         preferred_element_type=jnp.float32)
         for i in range(MS)], axis=0) * (1.0 / TEMP)

    def _iter(_, la):
        m2 = jnp.max(la, axis=1, keepdims=True)
        la = la - (m2 + jnp.log(jnp.sum(jnp.exp(la - m2), axis=1, keepdims=True)))
        m1 = jnp.max(la, axis=0, keepdims=True)
        la = la - (m1 + jnp.log(jnp.sum(jnp.exp(la - m1), axis=0, keepdims=True)))
        return la

    la3 = lax.fori_loop(0, SINK_ITERS, _iter, la3)
    plan3 = jnp.exp(la3)
    plan_pm = jnp.stack(
        [lax.dot_general(plan3[i], eye_ms, (((0,), (0,)), ((), ())),
                         preferred_element_type=jnp.float32)
         for i in range(MS)], axis=1)
    for b in range(PB):
        hc_b = xb[30 * b + MS:30 * b + 30, :]
        pc_ref[b] = jnp.dot(plan_pm[b], hc_b, preferred_element_type=jnp.float32)
    diff = jnp.maximum(hq_ref[...] - pc_ref[...], 0.0)
    r = jnp.sum(jnp.sum(diff, axis=2), axis=1)
    out[...] = (-r).reshape(1, 1, PB)


def _sinkhorn_scores(h, t):
    grid = (NP // PB,)
    return pl.pallas_call(
        _sinkhorn_body,
        grid=grid,
        in_specs=[
            pl.BlockSpec((PB * 30, D), lambda i: (i, 0)),
            pl.BlockSpec((PB * 30, TD), lambda i: (i, 0)),
        ],
        out_specs=pl.BlockSpec((1, 1, PB), lambda i: (i, 0, 0)),
        out_shape=jax.ShapeDtypeStruct((NP // PB, 1, PB), jnp.float32),
        scratch_shapes=[
            pltpu.VMEM((PB, MS, MS), jnp.float32),
            pltpu.VMEM((PB, MS, D), jnp.float32),
            pltpu.VMEM((PB, MS, D), jnp.float32),
        ],
    )(h, t)


def _sc_gather(table, idx2d):
    ni = idx2d.shape[1]
    cols = table.shape[1]
    mesh = plsc.VectorSubcoreMesh(core_axis_name="c", subcore_axis_name="s")

    @functools.partial(
        pl.kernel,
        out_type=jax.ShapeDtypeStruct((ni, cols), jnp.float32),
        mesh=mesh,
    )
    def k(tab_hbm, i_hbm, o_hbm):
        def body(i_vmem, o_vmem):
            pltpu.sync_copy(tab_hbm.at[i_vmem.at[0]], o_vmem)

        pltpu.emit_pipeline(
            body,
            grid=(ni // GW,),
            in_specs=[pl.BlockSpec((1, GW), lambda i: (0, i))],
            out_specs=[pl.BlockSpec((GW, cols), lambda i: (i, 0))],
            core_axis_name=("c", "s"),
            dimension_semantics=(pltpu.PARALLEL,),
        )(i_hbm, o_hbm)

    return k(table, idx2d)


def _sc_scatter_add(m2, idx2d, zeros_nd):
    nch = idx2d.shape[0]
    ch_per_core = nch // _SC_CORES
    ch_per_worker = nch // _SC_WORKERS
    rows_per_sub = N // _SC_SUBCORES
    mesh = plsc.VectorSubcoreMesh(core_axis_name="c", subcore_axis_name="s")

    @functools.partial(
        pl.kernel,
        out_type=jax.ShapeDtypeStruct((_SC_CORES, N, D), jnp.float32),
        mesh=mesh,
        scratch_types=[
            pltpu.VMEM_SHARED((N, D), jnp.float32),
            pltpu.VMEM((CW,), jnp.int32),
            pltpu.VMEM((CW, D), jnp.float32),
        ],
    )
    def k(m_hbm, i_hbm, z_hbm, o_hbm, acc_shared, idx_v, m_v):
        c = lax.axis_index("c")
        s = lax.axis_index("s")
        row0 = s * rows_per_sub
        pltpu.sync_copy(z_hbm.at[pl.ds(row0, rows_per_sub)],
                        acc_shared.at[pl.ds(row0, rows_per_sub)])
        plsc.subcore_barrier()
        base_chunk = c * ch_per_core + s * ch_per_worker

        @pl.loop(0, ch_per_worker)
        def _(j):
            ch = base_chunk + j
            pltpu.sync_copy(i_hbm.at[ch], idx_v)
            pltpu.sync_copy(m_hbm.at[pl.ds(ch * CW, CW)], m_v)
            pltpu.sync_copy(m_v, acc_shared.at[idx_v], add=True)

        plsc.subcore_barrier()
        pltpu.sync_copy(acc_shared.at[pl.ds(row0, rows_per_sub)],
                        o_hbm.at[c, pl.ds(row0, rows_per_sub)])

    return k(m2, idx2d, zeros_nd)


def kernel(node_features, edge_features, from_idx, to_idx,
           enc_node_W, enc_node_b, enc_edge_W, enc_edge_b,
           msg_W1, msg_b1, msg_W2, msg_b2,
           rmsg_W1, rmsg_b1, rmsg_W2, rmsg_b2,
           node_W, node_b, fc1_W, fc1_b, fc2_W, fc2_b):
    f32 = jnp.float32
    from_i = from_idx.astype(jnp.int32)
    to_i = to_idx.astype(jnp.int32)

    wf = jnp.concatenate([msg_W1[:D], rmsg_W1[D:2 * D]], axis=1)
    wt = jnp.concatenate([msg_W1[D:2 * D], rmsg_W1[:D]], axis=1)
    wft = jnp.concatenate([wf, wt], axis=0)
    wcc = jnp.concatenate([msg_W1[2 * D:], rmsg_W1[2 * D:]], axis=1)
    bcc = jnp.concatenate([msg_b1, rmsg_b1]).reshape(1, 2 * H)
    wcomb = enc_edge_W @ wcc
    bcomb = enc_edge_b.reshape(1, DE) @ wcc + bcc
    zh = jnp.zeros((H, H), f32)
    w2d = jnp.concatenate(
        [jnp.concatenate([msg_W2, zh], axis=1),
         jnp.concatenate([zh, rmsg_W2], axis=1)], axis=0)
    b2d = jnp.concatenate([msg_b2, rmsg_b2]).reshape(1, 2 * H)
    nwa = node_W[:D]
    nwb = node_W[D:]
    benc = enc_node_b.reshape(1, D)
    nb = node_b.reshape(1, D)
    fb1 = fc1_b.reshape(1, TD)
    fb2 = fc2_b.reshape(1, TD)

    gat_idx = jnp.concatenate([from_i, to_i]).reshape(1, E2)
    sct_idx = jnp.concatenate([to_i, from_i]).reshape(E2 // CW, CW)
    zeros_nd = jnp.zeros((N, D), f32)

    h = _encode(node_features, enc_node_W, benc)

    for _ in range(NLAYERS):
        g = _sc_gather(h, gat_idx)
        m = _messages(g.reshape(2, E, D), edge_features, wcomb, bcomb,
                      wft, w2d, b2d)
        parts = _sc_scatter_add(m.reshape(E2, H), sct_idx, zeros_nd)
        h = _update_final(h, parts, nwa, nwb, nb)

    t = _transform(h, fc1_W, fb1, fc2_W, fb2)
    scores = _sinkhorn_scores(h, t)
    return scores.reshape(NP)


# speedup vs baseline: 4.4343x; 1.1260x over previous
import functools

import jax
import jax.numpy as jnp
from jax import lax
from jax.experimental import pallas as pl
from jax.experimental.pallas import tpu as pltpu
from jax.experimental.pallas import tpu_sc as plsc

N = 7680
E = 30720
D = 128
DE = 16
H = 128
TD = 64
MS = 15
NP = 256
NLAYERS = 3
TEMP = 0.1
SINK_ITERS = 20

E2 = 2 * E
NODE_BLK = 512
EDGE_BLK = 2048
GW = 128
CW = 128
PB = 128

_SC_CORES = 2
_SC_SUBCORES = 16
_SC_WORKERS = _SC_CORES * _SC_SUBCORES


def _enc_body(nf, wenc, benc, h_out):
    h_out[...] = jnp.dot(nf[...], wenc[...], preferred_element_type=jnp.float32) + benc[...]


def _encode(nf, wenc, benc):
    grid = (N // NODE_BLK,)
    return pl.pallas_call(
        _enc_body,
        grid=grid,
        in_specs=[
            pl.BlockSpec((NODE_BLK, D), lambda i: (i, 0)),
            pl.BlockSpec((D, D), lambda i: (0, 0)),
            pl.BlockSpec((1, D), lambda i: (0, 0)),
        ],
        out_specs=pl.BlockSpec((NODE_BLK, D), lambda i: (i, 0)),
        out_shape=jax.ShapeDtypeStruct((N, D), jnp.float32),
    )(nf, wenc, benc)


def _msg_body(g, ef, wcomb, bcomb, wft, w2d, b2d, m_out):
    ec = jnp.dot(ef[...], wcomb[...], preferred_element_type=jnp.float32) + bcomb[...]
    gg = jnp.concatenate([g[0], g[1]], axis=1)
    u = jnp.dot(gg, wft[...], preferred_element_type=jnp.float32) + ec
    x = jnp.maximum(u, 0.0)
    m = jnp.dot(x, w2d[...], preferred_element_type=jnp.float32) + b2d[...]
    m_out[0] = m[:, :H]
    m_out[1] = m[:, H:]


def _messages(g3, ef, wcomb, bcomb, wft, w2d, b2d):
    grid = (E // EDGE_BLK,)
    return pl.pallas_call(
        _msg_body,
        grid=grid,
        in_specs=[
            pl.BlockSpec((2, EDGE_BLK, D), lambda i: (0, i, 0)),
            pl.BlockSpec((EDGE_BLK, DE), lambda i: (i, 0)),
            pl.BlockSpec((DE, 2 * H), lambda i: (0, 0)),
            pl.BlockSpec((1, 2 * H), lambda i: (0, 0)),
            pl.BlockSpec((2 * D, 2 * H), lambda i: (0, 0)),
            pl.BlockSpec((2 * H, 2 * H), lambda i: (0, 0)),
            pl.BlockSpec((1, 2 * H), lambda i: (0, 0)),
        ],
        out_specs=pl.BlockSpec((2, EDGE_BLK, H), lambda i: (0, i, 0)),
        out_shape=jax.ShapeDtypeStruct((2, E, H), jnp.float32),
    )(g3, ef, wcomb, bcomb, wft, w2d, b2d)


def _update_final_body(h, p, nwa, nwb, nb, h_out):
    agg = p[0] + p[1]
    h_out[...] = (jnp.dot(h[...], nwa[...], preferred_element_type=jnp.float32)
                  + jnp.dot(agg, nwb[...], preferred_element_type=jnp.float32)
                  + nb[...])


def _update_final(h, parts, nwa, nwb, nb):
    grid = (N // NODE_BLK,)
    return pl.pallas_call(
        _update_final_body,
        grid=grid,
        in_specs=[
            pl.BlockSpec((NODE_BLK, D), lambda i: (i, 0)),
            pl.BlockSpec((2, NODE_BLK, H), lambda i: (0, i, 0)),
            pl.BlockSpec((D, D), lambda i: (0, 0)),
            pl.BlockSpec((H, D), lambda i: (0, 0)),
            pl.BlockSpec((1, D), lambda i: (0, 0)),
        ],
        out_specs=pl.BlockSpec((NODE_BLK, D), lambda i: (i, 0)),
        out_shape=jax.ShapeDtypeStruct((N, D), jnp.float32),
    )(h, parts, nwa, nwb, nb)


def _transform_body(h, w1, b1, w2, b2, t_out):
    t = jnp.maximum(jnp.dot(h[...], w1[...], preferred_element_type=jnp.float32) + b1[...], 0.0)
    t_out[...] = jnp.dot(t, w2[...], preferred_element_type=jnp.float32) + b2[...]


def _transform(h, w1, b1, w2, b2):
    grid = (N // NODE_BLK,)
    return pl.pallas_call(
        _transform_body,
        grid=grid,
        in_specs=[
            pl.BlockSpec((NODE_BLK, D), lambda i: (i, 0)),
            pl.BlockSpec((D, TD), lambda i: (0, 0)),
            pl.BlockSpec((1, TD), lambda i: (0, 0)),
            pl.BlockSpec((TD, TD), lambda i: (0, 0)),
            pl.BlockSpec((1, TD), lambda i: (0, 0)),
        ],
        out_specs=pl.BlockSpec((NODE_BLK, TD), lambda i: (i, 0)),
        out_shape=jax.ShapeDtypeStruct((N, TD), jnp.float32),
    )(h, w1, b1, w2, b2)


def _sinkhorn_body(x, t, out, s_ref, pc_ref, hq_ref):
    xb = x[...]
    tb = t[...]
    for b in range(PB):
        tq_b = tb[30 * b:30 * b + MS, :]
        tc_b = tb[30 * b + MS:30 * b + 30, :]
        s_ref[b] = lax.dot_general(tq_b, tc_b, (((1,), (1,)), ((), ())),
                                   preferred_element_type=jnp.float32)
        hq_ref[b] = xb[30 * b:30 * b + MS, :]
    s_pm = s_ref[...]

    eye_pb = jnp.eye(PB, dtype=jnp.float32)
    eye_ms = jnp.eye(MS, dtype=jnp.float32)
    la3 = jnp.stack(
        [lax.dot_general(s_pm[:, i, :], eye_pb, (((0,), (0,)), ((), ())),
                         preferred_element_type=jnp.float32)
         for i in range(MS)], axis=0) * (1.0 / TEMP)

    def _iter(_, la):
        m2 = jnp.max(la, axis=1, keepdims=True)
        la = la - (m2 + jnp.log(jnp.sum(jnp.exp(la - m2), axis=1, keepdims=True)))
        m1 = jnp.max(la, axis=0, keepdims=True)
        la = la - (m1 + jnp.log(jnp.sum(jnp.exp(la - m1), axis=0, keepdims=True)))
        return la

    la3 = lax.fori_loop(0, SINK_ITERS, _iter, la3)
    plan3 = jnp.exp(la3)
    plan_pm = jnp.stack(
        [lax.dot_general(plan3[i], eye_ms, (((0,), (0,)), ((), ())),
                         preferred_element_type=jnp.float32)
         for i in range(MS)], axis=1)
    for b in range(PB):
        hc_b = xb[30 * b + MS:30 * b + 30, :]
        pc_ref[b] = jnp.dot(plan_pm[b], hc_b, preferred_element_type=jnp.float32)
    diff = jnp.maximum(hq_ref[...] - pc_ref[...], 0.0)
    r = jnp.sum(jnp.sum(diff, axis=2), axis=1)
    out[...] = (-r).reshape(1, 1, PB)


def _sinkhorn_scores(h, t):
    grid = (NP // PB,)
    return pl.pallas_call(
        _sinkhorn_body,
        grid=grid,
        in_specs=[
            pl.BlockSpec((PB * 30, D), lambda i: (i, 0)),
            pl.BlockSpec((PB * 30, TD), lambda i: (i, 0)),
        ],
        out_specs=pl.BlockSpec((1, 1, PB), lambda i: (i, 0, 0)),
        out_shape=jax.ShapeDtypeStruct((NP // PB, 1, PB), jnp.float32),
        scratch_shapes=[
            pltpu.VMEM((PB, MS, MS), jnp.float32),
            pltpu.VMEM((PB, MS, D), jnp.float32),
            pltpu.VMEM((PB, MS, D), jnp.float32),
        ],
    )(h, t)


def _sc_gather(table, idx2d):
    ni = idx2d.shape[1]
    cols = table.shape[1]
    mesh = plsc.VectorSubcoreMesh(core_axis_name="c", subcore_axis_name="s")

    @functools.partial(
        pl.kernel,
        out_type=jax.ShapeDtypeStruct((ni, cols), jnp.float32),
        mesh=mesh,
    )
    def k(tab_hbm, i_hbm, o_hbm):
        def body(i_vmem, o_vmem):
            pltpu.sync_copy(tab_hbm.at[i_vmem.at[0]], o_vmem)

        pltpu.emit_pipeline(
            body,
            grid=(ni // GW,),
            in_specs=[pl.BlockSpec((1, GW), lambda i: (0, i))],
            out_specs=[pl.BlockSpec((GW, cols), lambda i: (i, 0))],
            core_axis_name=("c", "s"),
            dimension_semantics=(pltpu.PARALLEL,),
        )(i_hbm, o_hbm)

    return k(table, idx2d)


def _sc_scatter_add(m2, idx2d, zeros_nd):
    nch = idx2d.shape[0] * idx2d.shape[1]
    ch_per_core = nch // _SC_CORES
    ch_per_worker = nch // _SC_WORKERS
    rows_per_sub = N // _SC_SUBCORES
    gch = 1
    ngroup = ch_per_worker // gch
    mesh = plsc.VectorSubcoreMesh(core_axis_name="c", subcore_axis_name="s")

    @functools.partial(
        pl.kernel,
        out_type=jax.ShapeDtypeStruct((_SC_CORES, N, D), jnp.float32),
        mesh=mesh,
        scratch_types=[
            pltpu.VMEM_SHARED((N, D), jnp.float32),
            pltpu.VMEM((ch_per_worker, CW), jnp.int32),
            pltpu.VMEM((gch * CW, D), jnp.float32),
            pltpu.VMEM((gch * CW, D), jnp.float32),
            pltpu.SemaphoreType.DMA,
            pltpu.SemaphoreType.DMA,
        ],
    )
    def k(m_hbm, i_hbm, z_hbm, o_hbm, acc_shared, idx_v, mb0, mb1, sem0, sem1):
        c = lax.axis_index("c")
        s = lax.axis_index("s")
        row0 = s * rows_per_sub
        wid = c * _SC_SUBCORES + s
        base_chunk = c * ch_per_core + s * ch_per_worker
        cp_init = pltpu.async_copy(z_hbm.at[pl.ds(row0, rows_per_sub)],
                                   acc_shared.at[pl.ds(row0, rows_per_sub)], sem1)
        pltpu.sync_copy(i_hbm.at[wid], idx_v)
        bufs = (mb0, mb1)
        sems = (sem0, sem1)
        cp = pltpu.async_copy(m_hbm.at[pl.ds(base_chunk * CW, gch * CW)], mb0, sem0)
        cp_init.wait()
        plsc.subcore_barrier()
        for g in range(ngroup):
            cp.wait()
            if g + 1 < ngroup:
                nxt = (base_chunk + (g + 1) * gch) * CW
                cp = pltpu.async_copy(m_hbm.at[pl.ds(nxt, gch * CW)],
                                      bufs[(g + 1) % 2], sems[(g + 1) % 2])
            buf = bufs[g % 2]
            for j in range(gch):
                pltpu.sync_copy(buf.at[pl.ds(j * CW, CW)],
                                acc_shared.at[idx_v.at[g * gch + j]], add=True)
        plsc.subcore_barrier()
        pltpu.sync_copy(acc_shared.at[pl.ds(row0, rows_per_sub)],
                        o_hbm.at[c, pl.ds(row0, rows_per_sub)])

    return k(m2, idx2d, zeros_nd)


def kernel(node_features, edge_features, from_idx, to_idx,
           enc_node_W, enc_node_b, enc_edge_W, enc_edge_b,
           msg_W1, msg_b1, msg_W2, msg_b2,
           rmsg_W1, rmsg_b1, rmsg_W2, rmsg_b2,
           node_W, node_b, fc1_W, fc1_b, fc2_W, fc2_b):
    f32 = jnp.float32
    from_i = from_idx.astype(jnp.int32)
    to_i = to_idx.astype(jnp.int32)

    wf = jnp.concatenate([msg_W1[:D], rmsg_W1[D:2 * D]], axis=1)
    wt = jnp.concatenate([msg_W1[D:2 * D], rmsg_W1[:D]], axis=1)
    wft = jnp.concatenate([wf, wt], axis=0)
    wcc = jnp.concatenate([msg_W1[2 * D:], rmsg_W1[2 * D:]], axis=1)
    bcc = jnp.concatenate([msg_b1, rmsg_b1]).reshape(1, 2 * H)
    wcomb = enc_edge_W @ wcc
    bcomb = enc_edge_b.reshape(1, DE) @ wcc + bcc
    zh = jnp.zeros((H, H), f32)
    w2d = jnp.concatenate(
        [jnp.concatenate([msg_W2, zh], axis=1),
         jnp.concatenate([zh, rmsg_W2], axis=1)], axis=0)
    b2d = jnp.concatenate([msg_b2, rmsg_b2]).reshape(1, 2 * H)
    nwa = node_W[:D]
    nwb = node_W[D:]
    benc = enc_node_b.reshape(1, D)
    nb = node_b.reshape(1, D)
    fb1 = fc1_b.reshape(1, TD)
    fb2 = fc2_b.reshape(1, TD)

    gat_idx = jnp.concatenate([from_i, to_i]).reshape(1, E2)
    sct_idx = jnp.concatenate([to_i, from_i]).reshape(_SC_WORKERS, E2 // (_SC_WORKERS * CW), CW)
    zeros_nd = jnp.zeros((N, D), f32)

    h = _encode(node_features, enc_node_W, benc)

    for _ in range(NLAYERS):
        g = _sc_gather(h, gat_idx)
        m = _messages(g.reshape(2, E, D), edge_features, wcomb, bcomb,
                      wft, w2d, b2d)
        parts = _sc_scatter_add(m.reshape(E2, H), sct_idx, zeros_nd)
        h = _update_final(h, parts, nwa, nwb, nb)

    t = _transform(h, fc1_W, fb1, fc2_W, fb2)
    scores = _sinkhorn_scores(h, t)
    return scores.reshape(NP)


# speedup vs baseline: 4.5805x; 1.0330x over previous
import functools

import jax
import jax.numpy as jnp
from jax import lax
from jax.experimental import pallas as pl
from jax.experimental.pallas import tpu as pltpu
from jax.experimental.pallas import tpu_sc as plsc

N = 7680
E = 30720
D = 128
DE = 16
H = 128
TD = 64
MS = 15
NP = 256
NLAYERS = 3
TEMP = 0.1
SINK_ITERS = 20

E2 = 2 * E
NODE_BLK = 512
EDGE_BLK = 2048
GW = 128
CW = 128
PB = 128

_SC_CORES = 2
_SC_SUBCORES = 16
_SC_WORKERS = _SC_CORES * _SC_SUBCORES


def _enc_body(nf, wenc, benc, h_out):
    h_out[...] = jnp.dot(nf[...], wenc[...], preferred_element_type=jnp.float32) + benc[...]


def _encode(nf, wenc, benc):
    grid = (N // NODE_BLK,)
    return pl.pallas_call(
        _enc_body,
        grid=grid,
        in_specs=[
            pl.BlockSpec((NODE_BLK, D), lambda i: (i, 0)),
            pl.BlockSpec((D, D), lambda i: (0, 0)),
            pl.BlockSpec((1, D), lambda i: (0, 0)),
        ],
        out_specs=pl.BlockSpec((NODE_BLK, D), lambda i: (i, 0)),
        out_shape=jax.ShapeDtypeStruct((N, D), jnp.float32),
    )(nf, wenc, benc)


def _msg_body(g, ef, wcomb, bcomb, wft, w2d, b2d, m_out):
    ec = jnp.dot(ef[...], wcomb[...], preferred_element_type=jnp.float32) + bcomb[...]
    gg = jnp.concatenate([g[0], g[1]], axis=1)
    u = jnp.dot(gg, wft[...], preferred_element_type=jnp.float32) + ec
    x = jnp.maximum(u, 0.0)
    m = jnp.dot(x, w2d[...], preferred_element_type=jnp.float32) + b2d[...]
    m_out[0] = m[:, :H]
    m_out[1] = m[:, H:]


def _messages(g3, ef, wcomb, bcomb, wft, w2d, b2d):
    grid = (E // EDGE_BLK,)
    return pl.pallas_call(
        _msg_body,
        grid=grid,
        in_specs=[
            pl.BlockSpec((2, EDGE_BLK, D), lambda i: (0, i, 0)),
            pl.BlockSpec((EDGE_BLK, DE), lambda i: (i, 0)),
            pl.BlockSpec((DE, 2 * H), lambda i: (0, 0)),
            pl.BlockSpec((1, 2 * H), lambda i: (0, 0)),
            pl.BlockSpec((2 * D, 2 * H), lambda i: (0, 0)),
            pl.BlockSpec((2 * H, 2 * H), lambda i: (0, 0)),
            pl.BlockSpec((1, 2 * H), lambda i: (0, 0)),
        ],
        out_specs=pl.BlockSpec((2, EDGE_BLK, H), lambda i: (0, i, 0)),
        out_shape=jax.ShapeDtypeStruct((2, E, H), jnp.float32),
    )(g3, ef, wcomb, bcomb, wft, w2d, b2d)


def _update_final_body(h, p, nwa, nwb, nb, h_out):
    agg = p[0] + p[1]
    h_out[...] = (jnp.dot(h[...], nwa[...], preferred_element_type=jnp.float32)
                  + jnp.dot(agg, nwb[...], preferred_element_type=jnp.float32)
                  + nb[...])


def _update_final(h, parts, nwa, nwb, nb):
    grid = (N // NODE_BLK,)
    return pl.pallas_call(
        _update_final_body,
        grid=grid,
        in_specs=[
            pl.BlockSpec((NODE_BLK, D), lambda i: (i, 0)),
            pl.BlockSpec((2, NODE_BLK, H), lambda i: (0, i, 0)),
            pl.BlockSpec((D, D), lambda i: (0, 0)),
            pl.BlockSpec((H, D), lambda i: (0, 0)),
            pl.BlockSpec((1, D), lambda i: (0, 0)),
        ],
        out_specs=pl.BlockSpec((NODE_BLK, D), lambda i: (i, 0)),
        out_shape=jax.ShapeDtypeStruct((N, D), jnp.float32),
    )(h, parts, nwa, nwb, nb)


def _sinkhorn_body(x, w1, b1, w2, b2, out, s_ref, pc_ref, hq_ref):
    xb = x[...]
    tt = jnp.maximum(jnp.dot(xb, w1[...], preferred_element_type=jnp.float32) + b1[...], 0.0)
    tb = jnp.dot(tt, w2[...], preferred_element_type=jnp.float32) + b2[...]
    for b in range(PB):
        tq_b = tb[30 * b:30 * b + MS, :]
        tc_b = tb[30 * b + MS:30 * b + 30, :]
        s_ref[b] = lax.dot_general(tq_b, tc_b, (((1,), (1,)), ((), ())),
                                   preferred_element_type=jnp.float32)
        hq_ref[b] = xb[30 * b:30 * b + MS, :]
    s_pm = s_ref[...]

    eye_pb = jnp.eye(PB, dtype=jnp.float32)
    eye_ms = jnp.eye(MS, dtype=jnp.float32)
    la3 = jnp.stack(
        [lax.dot_general(s_pm[:, i, :], eye_pb, (((0,), (0,)), ((), ())),
                         preferred_element_type=jnp.float32)
         for i in range(MS)], axis=0) * (1.0 / TEMP)

    def _iter(_, la):
        m2 = jnp.max(la, axis=1, keepdims=True)
        la = la - (m2 + jnp.log(jnp.sum(jnp.exp(la - m2), axis=1, keepdims=True)))
        m1 = jnp.max(la, axis=0, keepdims=True)
        la = la - (m1 + jnp.log(jnp.sum(jnp.exp(la - m1), axis=0, keepdims=True)))
        return la

    la3 = lax.fori_loop(0, SINK_ITERS, _iter, la3)
    plan3 = jnp.exp(la3)
    plan_pm = jnp.stack(
        [lax.dot_general(plan3[i], eye_ms, (((0,), (0,)), ((), ())),
                         preferred_element_type=jnp.float32)
         for i in range(MS)], axis=1)
    for b in range(PB):
        hc_b = xb[30 * b + MS:30 * b + 30, :]
        pc_ref[b] = jnp.dot(plan_pm[b], hc_b, preferred_element_type=jnp.float32)
    diff = jnp.maximum(hq_ref[...] - pc_ref[...], 0.0)
    r = jnp.sum(jnp.sum(diff, axis=2), axis=1)
    out[...] = (-r).reshape(1, 1, PB)


def _sinkhorn_scores(h, w1, b1, w2, b2):
    grid = (NP // PB,)
    return pl.pallas_call(
        _sinkhorn_body,
        grid=grid,
        in_specs=[
            pl.BlockSpec((PB * 30, D), lambda i: (i, 0)),
            pl.BlockSpec((D, TD), lambda i: (0, 0)),
            pl.BlockSpec((1, TD), lambda i: (0, 0)),
            pl.BlockSpec((TD, TD), lambda i: (0, 0)),
            pl.BlockSpec((1, TD), lambda i: (0, 0)),
        ],
        out_specs=pl.BlockSpec((1, 1, PB), lambda i: (i, 0, 0)),
        out_shape=jax.ShapeDtypeStruct((NP // PB, 1, PB), jnp.float32),
        scratch_shapes=[
            pltpu.VMEM((PB, MS, MS), jnp.float32),
            pltpu.VMEM((PB, MS, D), jnp.float32),
            pltpu.VMEM((PB, MS, D), jnp.float32),
        ],
    )(h, w1, b1, w2, b2)


def _sc_gather(table, idx2d):
    ni = idx2d.shape[1]
    cols = table.shape[1]
    mesh = plsc.VectorSubcoreMesh(core_axis_name="c", subcore_axis_name="s")

    @functools.partial(
        pl.kernel,
        out_type=jax.ShapeDtypeStruct((ni, cols), table.dtype),
        mesh=mesh,
    )
    def k(tab_hbm, i_hbm, o_hbm):
        def body(i_vmem, o_vmem):
            pltpu.sync_copy(tab_hbm.at[i_vmem.at[0]], o_vmem)

        pltpu.emit_pipeline(
            body,
            grid=(ni // GW,),
            in_specs=[pl.BlockSpec((1, GW), lambda i: (0, i))],
            out_specs=[pl.BlockSpec((GW, cols), lambda i: (i, 0))],
            core_axis_name=("c", "s"),
            dimension_semantics=(pltpu.PARALLEL,),
        )(i_hbm, o_hbm)

    return k(table, idx2d)


def _sc_scatter_add(m2, idx2d, zeros_nd):
    nch = idx2d.shape[0] * idx2d.shape[1]
    ch_per_core = nch // _SC_CORES
    ch_per_worker = nch // _SC_WORKERS
    rows_per_sub = N // _SC_SUBCORES
    gch = 1
    ngroup = ch_per_worker // gch
    mesh = plsc.VectorSubcoreMesh(core_axis_name="c", subcore_axis_name="s")

    @functools.partial(
        pl.kernel,
        out_type=jax.ShapeDtypeStruct((_SC_CORES, N, D), jnp.float32),
        mesh=mesh,
        scratch_types=[
            pltpu.VMEM_SHARED((N, D), jnp.float32),
            pltpu.VMEM((ch_per_worker, CW), jnp.int32),
            pltpu.VMEM((gch * CW, D), jnp.float32),
            pltpu.VMEM((gch * CW, D), jnp.float32),
            pltpu.SemaphoreType.DMA,
            pltpu.SemaphoreType.DMA,
        ],
    )
    def k(m_hbm, i_hbm, z_hbm, o_hbm, acc_shared, idx_v, mb0, mb1, sem0, sem1):
        c = lax.axis_index("c")
        s = lax.axis_index("s")
        row0 = s * rows_per_sub
        wid = c * _SC_SUBCORES + s
        base_chunk = c * ch_per_core + s * ch_per_worker
        cp_init = pltpu.async_copy(z_hbm.at[pl.ds(row0, rows_per_sub)],
                                   acc_shared.at[pl.ds(row0, rows_per_sub)], sem1)
        pltpu.sync_copy(i_hbm.at[wid], idx_v)
        bufs = (mb0, mb1)
        sems = (sem0, sem1)
        cp = pltpu.async_copy(m_hbm.at[pl.ds(base_chunk * CW, gch * CW)], mb0, sem0)
        cp_init.wait()
        plsc.subcore_barrier()
        for g in range(ngroup):
            cp.wait()
            if g + 1 < ngroup:
                nxt = (base_chunk + (g + 1) * gch) * CW
                cp = pltpu.async_copy(m_hbm.at[pl.ds(nxt, gch * CW)],
                                      bufs[(g + 1) % 2], sems[(g + 1) % 2])
            buf = bufs[g % 2]
            for j in range(gch):
                pltpu.sync_copy(buf.at[pl.ds(j * CW, CW)],
                                acc_shared.at[idx_v.at[g * gch + j]], add=True)
        plsc.subcore_barrier()
        pltpu.sync_copy(acc_shared.at[pl.ds(row0, rows_per_sub)],
                        o_hbm.at[c, pl.ds(row0, rows_per_sub)])

    return k(m2, idx2d, zeros_nd)


def kernel(node_features, edge_features, from_idx, to_idx,
           enc_node_W, enc_node_b, enc_edge_W, enc_edge_b,
           msg_W1, msg_b1, msg_W2, msg_b2,
           rmsg_W1, rmsg_b1, rmsg_W2, rmsg_b2,
           node_W, node_b, fc1_W, fc1_b, fc2_W, fc2_b):
    f32 = jnp.float32
    from_i = from_idx.astype(jnp.int32)
    to_i = to_idx.astype(jnp.int32)

    wf = jnp.concatenate([msg_W1[:D], rmsg_W1[D:2 * D]], axis=1)
    wt = jnp.concatenate([msg_W1[D:2 * D], rmsg_W1[:D]], axis=1)
    wft = jnp.concatenate([wf, wt], axis=0)
    wcc = jnp.concatenate([msg_W1[2 * D:], rmsg_W1[2 * D:]], axis=1)
    bcc = jnp.concatenate([msg_b1, rmsg_b1]).reshape(1, 2 * H)
    wcomb = enc_edge_W @ wcc
    bcomb = enc_edge_b.reshape(1, DE) @ wcc + bcc
    zh = jnp.zeros((H, H), f32)
    w2d = jnp.concatenate(
        [jnp.concatenate([msg_W2, zh], axis=1),
         jnp.concatenate([zh, rmsg_W2], axis=1)], axis=0)
    b2d = jnp.concatenate([msg_b2, rmsg_b2]).reshape(1, 2 * H)
    nwa = node_W[:D]
    nwb = node_W[D:]
    benc = enc_node_b.reshape(1, D)
    nb = node_b.reshape(1, D)
    fb1 = fc1_b.reshape(1, TD)
    fb2 = fc2_b.reshape(1, TD)

    gat_idx = jnp.concatenate([from_i, to_i]).reshape(1, E2)
    sct_idx = jnp.concatenate([to_i, from_i]).reshape(_SC_WORKERS, E2 // (_SC_WORKERS * CW), CW)
    zeros_nd = jnp.zeros((N, D), f32)

    h = _encode(node_features, enc_node_W, benc)

    for layer in range(NLAYERS):
        g = _sc_gather(h, gat_idx)
        m = _messages(g.reshape(2, E, D), edge_features, wcomb, bcomb,
                      wft, w2d, b2d)
        parts = _sc_scatter_add(m.reshape(E2, H), sct_idx, zeros_nd)
        h = _update_final(h, parts, nwa, nwb, nb)

    scores = _sinkhorn_scores(h, fc1_W, fb1, fc2_W, fb2)
    return scores.reshape(NP)
